# baseline (speedup 1.0000x reference)
import functools
import math

import jax
import jax.numpy as jnp
from jax.experimental import pallas as pl
from jax.experimental.pallas import tpu as pltpu

TEMPERATURE = 0.02
FILTER_THRESHOLD = 0.95
FILTER_FACTOR = 0.5
SCALE = 1.0 / TEMPERATURE
LOG2E = math.log2(math.e)
NEG_BIG = -1e30


def _body(q_ref, d_ref, dpos_ref, out_ref,
          qs_ref, s_ref, pos_ref, m_ref, l_ref, *, n_col_blocks, b_rows):
    c = pl.program_id(0)

    @pl.when(c == 0)
    def _init():
        q = q_ref[...]
        pos_ref[...] = (
            jnp.sum(q * dpos_ref[...], axis=1, keepdims=True)
            * (SCALE * LOG2E)
        )
        qs_ref[...] = (q * (SCALE * LOG2E)).astype(jnp.bfloat16)
        m_ref[...] = jnp.full((b_rows, 1), NEG_BIG, dtype=jnp.float32)
        l_ref[...] = jnp.zeros((b_rows, 1), dtype=jnp.float32)

    cur = jax.lax.rem(c, 2)
    prev = jax.lax.rem(c + 1, 2)
    s_ref[cur] = jax.lax.dot_general(
        qs_ref[...], d_ref[...].astype(jnp.bfloat16),
        dimension_numbers=(((1,), (1,)), ((), ())),
        preferred_element_type=jnp.float32,
    )

    valid = c >= 1
    s = s_ref[prev]
    thresh = FILTER_THRESHOLD * pos_ref[...]
    s = jnp.where(s > thresh, s * FILTER_FACTOR, s)

    m_prev = m_ref[...]
    bm = jnp.max(s, axis=1, keepdims=True)
    m_cur = jnp.maximum(m_prev, jnp.where(valid, bm, NEG_BIG))
    bsum = jnp.sum(jnp.exp2(s - m_cur), axis=1, keepdims=True)
    l_ref[...] = (
        l_ref[...] * jnp.exp2(m_prev - m_cur)
        + jnp.where(valid, bsum, 0.0)
    )
    m_ref[...] = m_cur

    @pl.when(c == n_col_blocks)
    def _final():
        pos = pos_ref[...]
        m_run = m_ref[...]
        l_run = l_ref[...]
        m_true = jnp.maximum(m_run, pos)
        corr = jnp.where(
            pos > 0.0,
            jnp.exp2(pos - m_true) - jnp.exp2(FILTER_FACTOR * pos - m_true),
            0.0,
        )
        l_true = l_run * jnp.exp2(m_run - m_true) + corr
        lse = m_true + jnp.log2(l_true)
        out_ref[...] = jnp.reshape(
            -jnp.sum(pos - lse) / (LOG2E * b_rows), (1, 1)
        )


def kernel(q_emb, d_emb, offset):
    b, k = q_emb.shape
    n = d_emb.shape[0]
    bn = 1024
    n_col_blocks = n // bn

    offset = jnp.asarray(offset, dtype=jnp.int32)
    d_pos = jax.lax.dynamic_slice(d_emb, (offset, 0), (b, k))

    body = functools.partial(_body, n_col_blocks=n_col_blocks, b_rows=b)
    last = n_col_blocks - 1
    out = pl.pallas_call(
        body,
        grid=(n_col_blocks + 1,),
        in_specs=[
            pl.BlockSpec((b, k), lambda c: (0, 0)),
            pl.BlockSpec((bn, k), lambda c: (jnp.minimum(c, last), 0)),
            pl.BlockSpec((b, k), lambda c: (0, 0)),
        ],
        out_specs=pl.BlockSpec((1, 1), lambda c: (0, 0)),
        out_shape=jax.ShapeDtypeStruct((1, 1), jnp.float32),
        scratch_shapes=[
            pltpu.VMEM((b, k), jnp.bfloat16),
            pltpu.VMEM((2, b, bn), jnp.float32),
            pltpu.VMEM((b, 1), jnp.float32),
            pltpu.VMEM((b, 1), jnp.float32),
            pltpu.VMEM((b, 1), jnp.float32),
        ],
    )(q_emb, d_emb, d_pos)
    return out[0, 0]


# speedup vs baseline: 1.3055x; 1.3055x over previous
import functools
import math

import jax
import jax.numpy as jnp
from jax.experimental import pallas as pl
from jax.experimental.pallas import tpu as pltpu

TEMPERATURE = 0.02
FILTER_THRESHOLD = 0.95
FILTER_FACTOR = 0.5
SCALE = 1.0 / TEMPERATURE
LOG2E = math.log2(math.e)
NEG_BIG = -1e30


def _epilogue(s_ref, pos_ref, m_ref, l_ref, valid):
    s = s_ref[...]
    thresh = FILTER_THRESHOLD * pos_ref[...]
    t = jnp.where(s > thresh, s * FILTER_FACTOR, s)
    bm = jnp.max(t, axis=1, keepdims=True)
    m_prev = m_ref[...]
    m_cur = jnp.maximum(m_prev, jnp.where(valid, bm, NEG_BIG))
    bsum = jnp.sum(jnp.exp2(t - m_cur), axis=1, keepdims=True)
    l_ref[...] = (
        l_ref[...] * jnp.exp2(m_prev - m_cur)
        + jnp.where(valid, bsum, 0.0)
    )
    m_ref[...] = m_cur


def _body(q_ref, d_ref, dpos_ref, out_ref,
          qs_ref, sa_ref, sb_ref, pos_ref, m_ref, l_ref,
          *, n_macro, bn, b_rows):
    c = pl.program_id(0)

    @pl.when(c == 0)
    def _init():
        q = q_ref[...]
        pos_ref[...] = (
            jnp.sum(q * dpos_ref[...], axis=1, keepdims=True)
            * (SCALE * LOG2E)
        )
        qs_ref[...] = q * (SCALE * LOG2E)
        m_ref[...] = jnp.full((b_rows, 1), NEG_BIG, dtype=jnp.float32)
        l_ref[...] = jnp.zeros((b_rows, 1), dtype=jnp.float32)

    qs = qs_ref[...]
    dims = (((1,), (1,)), ((), ()))

    sa_ref[...] = jax.lax.dot_general(
        qs, d_ref[0:bn, :], dimension_numbers=dims,
        preferred_element_type=jnp.float32,
    )
    _epilogue(sb_ref, pos_ref, m_ref, l_ref, valid=c >= 1)

    sb_ref[...] = jax.lax.dot_general(
        qs, d_ref[bn:2 * bn, :], dimension_numbers=dims,
        preferred_element_type=jnp.float32,
    )
    _epilogue(sa_ref, pos_ref, m_ref, l_ref, valid=c <= n_macro - 1)

    @pl.when(c == n_macro)
    def _final():
        pos = pos_ref[...]
        m_run = m_ref[...]
        l_run = l_ref[...]
        m_true = jnp.maximum(m_run, pos)
        corr = jnp.where(
            pos > 0.0,
            jnp.exp2(pos - m_true) - jnp.exp2(FILTER_FACTOR * pos - m_true),
            0.0,
        )
        l_true = l_run * jnp.exp2(m_run - m_true) + corr
        lse = m_true + jnp.log2(l_true)
        out_ref[...] = jnp.reshape(
            -jnp.sum(pos - lse) / (LOG2E * b_rows), (1, 1)
        )


def kernel(q_emb, d_emb, offset):
    b, k = q_emb.shape
    n = d_emb.shape[0]
    bn = 1024
    n_macro = n // (2 * bn)

    offset = jnp.asarray(offset, dtype=jnp.int32)
    d_pos = jax.lax.dynamic_slice(d_emb, (offset, 0), (b, k))

    body = functools.partial(_body, n_macro=n_macro, bn=bn, b_rows=b)
    last = n_macro - 1
    out = pl.pallas_call(
        body,
        grid=(n_macro + 1,),
        in_specs=[
            pl.BlockSpec((b, k), lambda c: (0, 0)),
            pl.BlockSpec((2 * bn, k), lambda c: (jnp.minimum(c, last), 0)),
            pl.BlockSpec((b, k), lambda c: (0, 0)),
        ],
        out_specs=pl.BlockSpec((1, 1), lambda c: (0, 0)),
        out_shape=jax.ShapeDtypeStruct((1, 1), jnp.float32),
        scratch_shapes=[
            pltpu.VMEM((b, k), jnp.float32),
            pltpu.VMEM((b, bn), jnp.float32),
            pltpu.VMEM((b, bn), jnp.float32),
            pltpu.VMEM((b, 1), jnp.float32),
            pltpu.VMEM((b, 1), jnp.float32),
            pltpu.VMEM((b, 1), jnp.float32),
        ],
    )(q_emb, d_emb, d_pos)
    return out[0, 0]


# speedup vs baseline: 1.5968x; 1.2231x over previous
import functools
import math

import jax
import jax.numpy as jnp
from jax.experimental import pallas as pl
from jax.experimental.pallas import tpu as pltpu

TEMPERATURE = 0.02
FILTER_THRESHOLD = 0.95
FILTER_FACTOR = 0.5
SCALE = 1.0 / TEMPERATURE
LOG2E = math.log2(math.e)
NEG_BIG = -1e30


def _epilogue(s_ref, pos_ref, m_ref, l_ref, valid):
    s = s_ref[...]
    thresh = FILTER_THRESHOLD * pos_ref[...]
    t = jnp.where(s > thresh, s * FILTER_FACTOR, s)
    bm = jnp.max(t, axis=1, keepdims=True)
    m_prev = m_ref[...]
    m_cur = jnp.maximum(m_prev, jnp.where(valid, bm, NEG_BIG))
    bsum = jnp.sum(jnp.exp2(t - m_cur), axis=1, keepdims=True)
    l_ref[...] = (
        l_ref[...] * jnp.exp2(m_prev - m_cur)
        + jnp.where(valid, bsum, 0.0)
    )
    m_ref[...] = m_cur


def _body(off_ref, q_ref, d_ref, dany_ref, out_ref,
          qs_ref, dpos_ref, sa_ref, sb_ref, pos_ref, m_ref, l_ref, sem,
          *, n_macro, bn, b_rows):
    c = pl.program_id(0)

    @pl.when(c == 0)
    def _init():
        off = pl.multiple_of(off_ref[0], 8)
        cp = pltpu.make_async_copy(
            dany_ref.at[pl.ds(off, b_rows), :], dpos_ref, sem
        )
        cp.start()
        q = q_ref[...]
        qs_ref[...] = q * (SCALE * LOG2E)
        cp.wait()
        pos_ref[...] = (
            jnp.sum(q * dpos_ref[...], axis=1, keepdims=True)
            * (SCALE * LOG2E)
        )
        m_ref[...] = jnp.full((b_rows, 1), NEG_BIG, dtype=jnp.float32)
        l_ref[...] = jnp.zeros((b_rows, 1), dtype=jnp.float32)

    qs = qs_ref[...]
    dims = (((1,), (1,)), ((), ()))

    sa_ref[...] = jax.lax.dot_general(
        qs, d_ref[0:bn, :], dimension_numbers=dims,
        preferred_element_type=jnp.float32,
    )
    _epilogue(sb_ref, pos_ref, m_ref, l_ref, valid=c >= 1)

    sb_ref[...] = jax.lax.dot_general(
        qs, d_ref[bn:2 * bn, :], dimension_numbers=dims,
        preferred_element_type=jnp.float32,
    )
    _epilogue(sa_ref, pos_ref, m_ref, l_ref, valid=True)

    @pl.when(c == n_macro - 1)
    def _final():
        _epilogue(sb_ref, pos_ref, m_ref, l_ref, valid=True)
        pos = pos_ref[...]
        m_run = m_ref[...]
        l_run = l_ref[...]
        m_true = jnp.maximum(m_run, pos)
        corr = jnp.where(
            pos > 0.0,
            jnp.exp2(pos - m_true) - jnp.exp2(FILTER_FACTOR * pos - m_true),
            0.0,
        )
        l_true = l_run * jnp.exp2(m_run - m_true) + corr
        lse = m_true + jnp.log2(l_true)
        out_ref[...] = jnp.reshape(
            -jnp.sum(pos - lse) / (LOG2E * b_rows), (1, 1)
        )


def kernel(q_emb, d_emb, offset):
    b, k = q_emb.shape
    n = d_emb.shape[0]
    bn = 1024
    n_macro = n // (2 * bn)

    offset = jnp.asarray(offset, dtype=jnp.int32).reshape((1,))

    body = functools.partial(_body, n_macro=n_macro, bn=bn, b_rows=b)
    out = pl.pallas_call(
        body,
        grid=(n_macro,),
        in_specs=[
            pl.BlockSpec(memory_space=pltpu.SMEM),
            pl.BlockSpec((b, k), lambda c: (0, 0)),
            pl.BlockSpec((2 * bn, k), lambda c: (c, 0)),
            pl.BlockSpec(memory_space=pl.ANY),
        ],
        out_specs=pl.BlockSpec((1, 1), lambda c: (0, 0)),
        out_shape=jax.ShapeDtypeStruct((1, 1), jnp.float32),
        scratch_shapes=[
            pltpu.VMEM((b, k), jnp.float32),
            pltpu.VMEM((b, k), jnp.float32),
            pltpu.VMEM((b, bn), jnp.float32),
            pltpu.VMEM((b, bn), jnp.float32),
            pltpu.VMEM((b, 1), jnp.float32),
            pltpu.VMEM((b, 1), jnp.float32),
            pltpu.VMEM((b, 1), jnp.float32),
            pltpu.SemaphoreType.DMA,
        ],
    )(offset, q_emb, d_emb, d_emb)
    return out[0, 0]
